# Initial kernel scaffold; baseline (speedup 1.0000x reference)
import jax
import jax.numpy as jnp
from jax.experimental import pallas as pl


def kernel(feat, edge_index, edge_weight, W, b):
    raise NotImplementedError("write your pallas kernel here")


# speedup vs baseline: 6.7402x; 6.7402x over previous
import functools

import jax
import jax.numpy as jnp
from jax import lax
from jax.experimental import pallas as pl
from jax.experimental.pallas import tpu as pltpu
from jax.experimental.pallas import tpu_sc as plsc

N_NODES = 10000
N_EDGES = 320000
F = 128

NC = 2
NS = 16
NW = NC * NS
EB = 80
NB_TOT = N_EDGES // EB
NB_W = NB_TOT // NW
N_PAD = 10240
ROWS_T = N_PAD // NS

_mesh = plsc.VectorSubcoreMesh(core_axis_name="c", subcore_axis_name="s")


@functools.partial(
    pl.kernel,
    out_type=(
        jax.ShapeDtypeStruct((N_PAD,), jnp.float32),
        jax.ShapeDtypeStruct((N_PAD,), jnp.float32),
        jax.ShapeDtypeStruct((N_PAD,), jnp.float32),
        jax.ShapeDtypeStruct((N_PAD,), jnp.float32),
    ),
    mesh=_mesh,
    scratch_types=[
        pltpu.VMEM((NB_W, EB), jnp.int32),
        pltpu.VMEM((NB_W, EB), jnp.int32),
        pltpu.VMEM((640,), jnp.float32),
        pltpu.VMEM_SHARED((N_PAD,), jnp.float32),
        pltpu.VMEM_SHARED((N_PAD,), jnp.float32),
    ],
)
def _sc_degrees(src_hbm, dst_hbm, og0_hbm, ig0_hbm, og1_hbm, ig1_hbm,
                src_v, dst_v, buf_v, og_s, ig_s):
    c = lax.axis_index("c")
    s = lax.axis_index("s")
    wid = c * NS + s

    pltpu.sync_copy(src_hbm.at[wid], src_v)
    pltpu.sync_copy(dst_hbm.at[wid], dst_v)

    def _fill(i, _, val):
        buf_v[pl.ds(i * 16, 16)] = jnp.full((16,), val, jnp.float32)
        return 0
    lax.fori_loop(0, 40, functools.partial(_fill, val=0.0), 0)
    lo = s * 640
    pltpu.sync_copy(buf_v, og_s.at[pl.ds(lo, 640)])
    pltpu.sync_copy(buf_v, ig_s.at[pl.ds(lo, 640)])
    plsc.subcore_barrier()

    lax.fori_loop(0, 40, functools.partial(_fill, val=1.0), 0)

    def _hist(j, _):
        ones_sl = buf_v.at[pl.ds(0, EB)]
        pltpu.sync_copy(ones_sl, og_s.at[src_v.at[j]], add=True)
        pltpu.sync_copy(ones_sl, ig_s.at[dst_v.at[j]], add=True)
        return 0
    lax.fori_loop(0, NB_W, _hist, 0)
    plsc.subcore_barrier()

    @pl.when(c == 0)
    def _():
        pltpu.sync_copy(og_s.at[pl.ds(lo, 640)], og0_hbm.at[pl.ds(lo, 640)])
        pltpu.sync_copy(ig_s.at[pl.ds(lo, 640)], ig0_hbm.at[pl.ds(lo, 640)])

    @pl.when(c == 1)
    def _():
        pltpu.sync_copy(og_s.at[pl.ds(lo, 640)], og1_hbm.at[pl.ds(lo, 640)])
        pltpu.sync_copy(ig_s.at[pl.ds(lo, 640)], ig1_hbm.at[pl.ds(lo, 640)])


CH = 25
NCH = NB_W // CH


@functools.partial(
    pl.kernel,
    out_type=jax.ShapeDtypeStruct((NC, N_PAD, F), jnp.float32),
    mesh=_mesh,
    scratch_types=[
        pltpu.VMEM((CH, EB), jnp.int32),
        pltpu.VMEM((CH, EB), jnp.int32),
        pltpu.VMEM((CH, EB), jnp.float32),
        pltpu.VMEM((EB, F), jnp.float32),
        pltpu.VMEM_SHARED((N_PAD, F), jnp.float32),
        pltpu.SemaphoreType.DMA,
    ],
)
def _sc_scatter(fs_hbm, src_hbm, dst_hbm, w_hbm, part_hbm,
                src_v, dst_v, w_v, rows_v, acc_s, sem):
    c = lax.axis_index("c")
    s = lax.axis_index("s")
    wid = c * NS + s

    def _zrow(r, _):
        for cc in range(F // 16):
            rows_v[r, pl.ds(cc * 16, 16)] = jnp.zeros((16,), jnp.float32)
        return 0
    lax.fori_loop(0, EB, _zrow, 0)
    lo = s * ROWS_T
    for k in range(ROWS_T // EB):
        pltpu.sync_copy(rows_v, acc_s.at[pl.ds(lo + k * EB, EB), :])
    plsc.subcore_barrier()

    def _chunk(k, _):
        pltpu.sync_copy(src_hbm.at[wid, k], src_v)
        pltpu.sync_copy(dst_hbm.at[wid, k], dst_v)
        pltpu.sync_copy(w_hbm.at[wid, k], w_v)

        def _edge_batch(j, _):
            pltpu.async_copy(fs_hbm.at[src_v.at[j]], rows_v, sem).wait()

            def _scale(g, _):
                wvec = w_v[j, pl.ds(g * 16, 16)]
                for e in range(16):
                    w = wvec[e]
                    r = g * 16 + e
                    for cc in range(F // 16):
                        sl = pl.ds(cc * 16, 16)
                        rows_v[r, sl] = rows_v[r, sl] * w
                return 0
            lax.fori_loop(0, EB // 16, _scale, 0)
            pltpu.sync_copy(rows_v, acc_s.at[dst_v.at[j]], add=True)
            return 0
        lax.fori_loop(0, CH, _edge_batch, 0)
        return 0
    lax.fori_loop(0, NCH, _chunk, 0)
    plsc.subcore_barrier()

    pltpu.sync_copy(acc_s.at[pl.ds(lo, ROWS_T), :],
                    part_hbm.at[c, pl.ds(lo, ROWS_T), :])


def _prescale_body(feat_ref, og_ref, out_ref):
    d = og_ref[0, :, 0] + og_ref[1, :, 0]
    ns = lax.rsqrt(jnp.maximum(d, 1.0))
    out_ref[:, :] = feat_ref[:, :] * ns[:, None]


def _tc_prescale(feat, og):
    blk = 1000
    return pl.pallas_call(
        _prescale_body,
        grid=(N_NODES // blk,),
        in_specs=[
            pl.BlockSpec((blk, F), lambda i: (i, 0)),
            pl.BlockSpec((NC, blk, 1), lambda i: (0, i, 0)),
        ],
        out_specs=pl.BlockSpec((blk, F), lambda i: (i, 0)),
        out_shape=jax.ShapeDtypeStruct((N_NODES, F), jnp.float32),
    )(feat, og)


def _final_body(p0_ref, p1_ref, w_ref, b_ref, ig_ref, out_ref):
    acc = p0_ref[:, :] + p1_ref[:, :]
    r = jnp.dot(acc, w_ref[:, :], preferred_element_type=jnp.float32)
    d = ig_ref[0, :, 0] + ig_ref[1, :, 0]
    nd = lax.rsqrt(jnp.maximum(d, 1.0))
    out_ref[:, :] = r * nd[:, None] + b_ref[:, :]


def _tc_final(p0, p1, W, b, ig):
    blk = 1000
    return pl.pallas_call(
        _final_body,
        grid=(N_NODES // blk,),
        in_specs=[
            pl.BlockSpec((blk, F), lambda i: (i, 0)),
            pl.BlockSpec((blk, F), lambda i: (i, 0)),
            pl.BlockSpec((F, F), lambda i: (0, 0)),
            pl.BlockSpec((1, F), lambda i: (0, 0)),
            pl.BlockSpec((NC, blk, 1), lambda i: (0, i, 0)),
        ],
        out_specs=pl.BlockSpec((blk, F), lambda i: (i, 0)),
        out_shape=jax.ShapeDtypeStruct((N_NODES, F), jnp.float32),
    )(p0, p1, W, b, ig)


@jax.jit
def kernel(feat, edge_index, edge_weight, W, b):
    src = edge_index[0].astype(jnp.int32)
    dst = edge_index[1].astype(jnp.int32)
    w2 = edge_weight

    og0, ig0, og1, ig1 = _sc_degrees(src.reshape(NW, NB_W, EB),
                                     dst.reshape(NW, NB_W, EB))
    og = jnp.stack([og0[:N_NODES], og1[:N_NODES]]).reshape(NC, N_NODES, 1)
    ig = jnp.stack([ig0[:N_NODES], ig1[:N_NODES]]).reshape(NC, N_NODES, 1)

    fs = _tc_prescale(feat, og)
    part = _sc_scatter(fs,
                       src.reshape(NW, NCH, CH, EB),
                       dst.reshape(NW, NCH, CH, EB),
                       w2.reshape(NW, NCH, CH, EB))
    return _tc_final(part[0, :N_NODES], part[1, :N_NODES], W,
                     b.reshape(1, F), ig)


# speedup vs baseline: 9.2949x; 1.3790x over previous
import functools

import jax
import jax.numpy as jnp
from jax import lax
from jax.experimental import pallas as pl
from jax.experimental.pallas import tpu as pltpu
from jax.experimental.pallas import tpu_sc as plsc

N_NODES = 10000
N_EDGES = 320000
F = 128

NC = 2
NS = 16
NW = NC * NS
EB = 80
NB_TOT = N_EDGES // EB
NB_W = NB_TOT // NW
N_PAD = 10240
ROWS_T = N_PAD // NS

_mesh = plsc.VectorSubcoreMesh(core_axis_name="c", subcore_axis_name="s")


@functools.partial(
    pl.kernel,
    out_type=(
        jax.ShapeDtypeStruct((N_PAD,), jnp.float32),
        jax.ShapeDtypeStruct((N_PAD,), jnp.float32),
        jax.ShapeDtypeStruct((N_PAD,), jnp.float32),
        jax.ShapeDtypeStruct((N_PAD,), jnp.float32),
    ),
    mesh=_mesh,
    scratch_types=[
        pltpu.VMEM((NB_W, EB), jnp.int32),
        pltpu.VMEM((NB_W, EB), jnp.int32),
        pltpu.VMEM((640,), jnp.float32),
        pltpu.VMEM_SHARED((N_PAD,), jnp.float32),
        pltpu.VMEM_SHARED((N_PAD,), jnp.float32),
    ],
)
def _sc_degrees(src_hbm, dst_hbm, og0_hbm, ig0_hbm, og1_hbm, ig1_hbm,
                src_v, dst_v, buf_v, og_s, ig_s):
    c = lax.axis_index("c")
    s = lax.axis_index("s")
    wid = c * NS + s

    pltpu.sync_copy(src_hbm.at[wid], src_v)
    pltpu.sync_copy(dst_hbm.at[wid], dst_v)

    def _fill(i, _, val):
        buf_v[pl.ds(i * 16, 16)] = jnp.full((16,), val, jnp.float32)
        return 0
    lax.fori_loop(0, 40, functools.partial(_fill, val=0.0), 0)
    lo = s * 640
    pltpu.sync_copy(buf_v, og_s.at[pl.ds(lo, 640)])
    pltpu.sync_copy(buf_v, ig_s.at[pl.ds(lo, 640)])
    plsc.subcore_barrier()

    lax.fori_loop(0, 40, functools.partial(_fill, val=1.0), 0)

    def _hist(j, _):
        ones_sl = buf_v.at[pl.ds(0, EB)]
        pltpu.sync_copy(ones_sl, og_s.at[src_v.at[j]], add=True)
        pltpu.sync_copy(ones_sl, ig_s.at[dst_v.at[j]], add=True)
        return 0
    lax.fori_loop(0, NB_W, _hist, 0)
    plsc.subcore_barrier()

    @pl.when(c == 0)
    def _():
        pltpu.sync_copy(og_s.at[pl.ds(lo, 640)], og0_hbm.at[pl.ds(lo, 640)])
        pltpu.sync_copy(ig_s.at[pl.ds(lo, 640)], ig0_hbm.at[pl.ds(lo, 640)])

    @pl.when(c == 1)
    def _():
        pltpu.sync_copy(og_s.at[pl.ds(lo, 640)], og1_hbm.at[pl.ds(lo, 640)])
        pltpu.sync_copy(ig_s.at[pl.ds(lo, 640)], ig1_hbm.at[pl.ds(lo, 640)])


CH = 25
NCH = NB_W // CH


@functools.partial(
    pl.kernel,
    out_type=jax.ShapeDtypeStruct((NC, N_PAD, F), jnp.float32),
    mesh=_mesh,
    scratch_types=[
        pltpu.VMEM((CH, EB), jnp.int32),
        pltpu.VMEM((CH, EB), jnp.int32),
        pltpu.VMEM((CH, EB), jnp.float32),
        pltpu.VMEM((EB, F), jnp.float32),
        pltpu.VMEM((EB, F), jnp.float32),
        pltpu.VMEM_SHARED((N_PAD, F), jnp.float32),
        pltpu.SemaphoreType.DMA,
        pltpu.SemaphoreType.DMA,
        pltpu.SemaphoreType.DMA,
        pltpu.SemaphoreType.DMA,
    ],
)
def _sc_scatter(fs_hbm, src_hbm, dst_hbm, w_hbm, part_hbm,
                src_v, dst_v, w_v, rows_a, rows_b, acc_s,
                sga, sgb, ssa, ssb):
    c = lax.axis_index("c")
    s = lax.axis_index("s")
    wid = c * NS + s

    def _zrow(r, _):
        for cc in range(F // 16):
            rows_a[r, pl.ds(cc * 16, 16)] = jnp.zeros((16,), jnp.float32)
        return 0
    lax.fori_loop(0, EB, _zrow, 0)
    lo = s * ROWS_T
    for k in range(ROWS_T // EB):
        pltpu.sync_copy(rows_a, acc_s.at[pl.ds(lo + k * EB, EB), :])
    plsc.subcore_barrier()

    def _gather(j, rows, sem):
        pltpu.make_async_copy(fs_hbm.at[src_v.at[j]], rows, sem).start()

    def _gather_wait(j, rows, sem):
        pltpu.make_async_copy(fs_hbm.at[src_v.at[j]], rows, sem).wait()

    def _scatter(j, rows, sem):
        pltpu.make_async_copy(rows, acc_s.at[dst_v.at[j]], sem).start(add=True)

    def _scatter_wait(j, rows, sem):
        pltpu.make_async_copy(rows, acc_s.at[dst_v.at[j]], sem).wait()

    def _scale(j, rows):
        def _grp(g, _):
            wvec = w_v[j, pl.ds(g * 16, 16)]
            for e in range(16):
                w = wvec[e]
                r = g * 16 + e
                for cc in range(F // 16):
                    sl = pl.ds(cc * 16, 16)
                    rows[r, sl] = rows[r, sl] * w
            return 0
        lax.fori_loop(0, EB // 16, _grp, 0)

    def _chunk(k, _):
        pltpu.sync_copy(src_hbm.at[wid, k], src_v)
        pltpu.sync_copy(dst_hbm.at[wid, k], dst_v)
        pltpu.sync_copy(w_hbm.at[wid, k], w_v)

        _gather(0, rows_a, sga)
        _gather(1, rows_b, sgb)

        def _pair(p, _):
            j0 = 2 * p
            j1 = 2 * p + 1
            _gather_wait(j0, rows_a, sga)
            _scale(j0, rows_a)
            _scatter(j0, rows_a, ssa)
            _gather_wait(j1, rows_b, sgb)
            _scale(j1, rows_b)
            _scatter(j1, rows_b, ssb)

            @pl.when(j0 + 2 < CH)
            def _():
                _scatter_wait(j0, rows_a, ssa)
                _gather(j0 + 2, rows_a, sga)

            @pl.when(j1 + 2 < CH)
            def _():
                _scatter_wait(j1, rows_b, ssb)
                _gather(j1 + 2, rows_b, sgb)
            return 0
        lax.fori_loop(0, CH // 2, _pair, 0)

        jl = CH - 1
        _gather_wait(jl, rows_a, sga)
        _scale(jl, rows_a)
        _scatter(jl, rows_a, ssa)
        _scatter_wait(jl, rows_a, ssa)
        _scatter_wait(jl, rows_b, ssb)
        return 0
    lax.fori_loop(0, NCH, _chunk, 0)
    plsc.subcore_barrier()

    pltpu.sync_copy(acc_s.at[pl.ds(lo, ROWS_T), :],
                    part_hbm.at[c, pl.ds(lo, ROWS_T), :])


def _prescale_body(feat_ref, og0_ref, og1_ref, out_ref):
    d = og0_ref[:, 0] + og1_ref[:, 0]
    ns = lax.rsqrt(jnp.maximum(d, 1.0))
    out_ref[:, :] = feat_ref[:, :] * ns[:, None]


def _tc_prescale(feat, og0, og1):
    blk = 1000
    return pl.pallas_call(
        _prescale_body,
        grid=(N_NODES // blk,),
        in_specs=[
            pl.BlockSpec((blk, F), lambda i: (i, 0)),
            pl.BlockSpec((blk, 1), lambda i: (i, 0)),
            pl.BlockSpec((blk, 1), lambda i: (i, 0)),
        ],
        out_specs=pl.BlockSpec((blk, F), lambda i: (i, 0)),
        out_shape=jax.ShapeDtypeStruct((N_NODES, F), jnp.float32),
    )(feat, og0.reshape(N_PAD, 1), og1.reshape(N_PAD, 1))


def _final_body(p_ref, w_ref, b_ref, ig0_ref, ig1_ref, out_ref):
    acc = p_ref[0, :, :] + p_ref[1, :, :]
    r = jnp.dot(acc, w_ref[:, :], preferred_element_type=jnp.float32)
    d = ig0_ref[:, 0] + ig1_ref[:, 0]
    nd = lax.rsqrt(jnp.maximum(d, 1.0))
    out_ref[:, :] = r * nd[:, None] + b_ref[:, :]


def _tc_final(part, W, b, ig0, ig1):
    blk = 1000
    return pl.pallas_call(
        _final_body,
        grid=(N_NODES // blk,),
        in_specs=[
            pl.BlockSpec((NC, blk, F), lambda i: (0, i, 0)),
            pl.BlockSpec((F, F), lambda i: (0, 0)),
            pl.BlockSpec((1, F), lambda i: (0, 0)),
            pl.BlockSpec((blk, 1), lambda i: (i, 0)),
            pl.BlockSpec((blk, 1), lambda i: (i, 0)),
        ],
        out_specs=pl.BlockSpec((blk, F), lambda i: (i, 0)),
        out_shape=jax.ShapeDtypeStruct((N_NODES, F), jnp.float32),
    )(part, W, b, ig0.reshape(N_PAD, 1), ig1.reshape(N_PAD, 1))


@jax.jit
def kernel(feat, edge_index, edge_weight, W, b):
    src = edge_index[0].astype(jnp.int32)
    dst = edge_index[1].astype(jnp.int32)
    w2 = edge_weight

    og0, ig0, og1, ig1 = _sc_degrees(src.reshape(NW, NB_W, EB),
                                     dst.reshape(NW, NB_W, EB))

    fs = _tc_prescale(feat, og0, og1)
    part = _sc_scatter(fs,
                       src.reshape(NW, NCH, CH, EB),
                       dst.reshape(NW, NCH, CH, EB),
                       w2.reshape(NW, NCH, CH, EB))
    return _tc_final(part, W, b.reshape(1, F), ig0, ig1)


# speedup vs baseline: 9.5901x; 1.0318x over previous
import functools

import jax
import jax.numpy as jnp
from jax import lax
from jax.experimental import pallas as pl
from jax.experimental.pallas import tpu as pltpu
from jax.experimental.pallas import tpu_sc as plsc

N_NODES = 10000
N_EDGES = 320000
F = 128

NC = 2
NS = 16
NW = NC * NS
EB = 80
NB_TOT = N_EDGES // EB
NB_W = NB_TOT // NW
N_PAD = 10240
ROWS_T = N_PAD // NS

_mesh = plsc.VectorSubcoreMesh(core_axis_name="c", subcore_axis_name="s")


@functools.partial(
    pl.kernel,
    out_type=(
        jax.ShapeDtypeStruct((N_PAD,), jnp.float32),
        jax.ShapeDtypeStruct((N_PAD,), jnp.float32),
        jax.ShapeDtypeStruct((N_PAD,), jnp.float32),
        jax.ShapeDtypeStruct((N_PAD,), jnp.float32),
    ),
    mesh=_mesh,
    scratch_types=[
        pltpu.VMEM((NB_W, EB), jnp.int32),
        pltpu.VMEM((NB_W, EB), jnp.int32),
        pltpu.VMEM((640,), jnp.float32),
        pltpu.VMEM_SHARED((N_PAD,), jnp.float32),
        pltpu.VMEM_SHARED((N_PAD,), jnp.float32),
        pltpu.SemaphoreType.DMA,
        pltpu.SemaphoreType.DMA,
    ],
)
def _sc_degrees(src_hbm, dst_hbm, og0_hbm, ig0_hbm, og1_hbm, ig1_hbm,
                src_v, dst_v, buf_v, og_s, ig_s, sem_o, sem_i):
    c = lax.axis_index("c")
    s = lax.axis_index("s")
    wid = c * NS + s

    pltpu.sync_copy(src_hbm.at[wid], src_v)
    pltpu.sync_copy(dst_hbm.at[wid], dst_v)

    def _fill(i, _, val):
        buf_v[pl.ds(i * 16, 16)] = jnp.full((16,), val, jnp.float32)
        return 0
    lax.fori_loop(0, 40, functools.partial(_fill, val=0.0), 0)
    lo = s * 640
    pltpu.sync_copy(buf_v, og_s.at[pl.ds(lo, 640)])
    pltpu.sync_copy(buf_v, ig_s.at[pl.ds(lo, 640)])
    plsc.subcore_barrier()

    lax.fori_loop(0, 40, functools.partial(_fill, val=1.0), 0)

    ones_sl = buf_v.at[pl.ds(0, EB)]

    def _hist(j, _):
        @pl.when(j > 0)
        def _():
            pltpu.make_async_copy(ones_sl, og_s.at[src_v.at[j]], sem_o).wait()
            pltpu.make_async_copy(ones_sl, ig_s.at[dst_v.at[j]], sem_i).wait()
        pltpu.make_async_copy(ones_sl, og_s.at[src_v.at[j]], sem_o).start(add=True)
        pltpu.make_async_copy(ones_sl, ig_s.at[dst_v.at[j]], sem_i).start(add=True)
        return 0
    lax.fori_loop(0, NB_W, _hist, 0)
    pltpu.make_async_copy(ones_sl, og_s.at[src_v.at[0]], sem_o).wait()
    pltpu.make_async_copy(ones_sl, ig_s.at[dst_v.at[0]], sem_i).wait()
    plsc.subcore_barrier()

    @pl.when(c == 0)
    def _():
        pltpu.sync_copy(og_s.at[pl.ds(lo, 640)], og0_hbm.at[pl.ds(lo, 640)])
        pltpu.sync_copy(ig_s.at[pl.ds(lo, 640)], ig0_hbm.at[pl.ds(lo, 640)])

    @pl.when(c == 1)
    def _():
        pltpu.sync_copy(og_s.at[pl.ds(lo, 640)], og1_hbm.at[pl.ds(lo, 640)])
        pltpu.sync_copy(ig_s.at[pl.ds(lo, 640)], ig1_hbm.at[pl.ds(lo, 640)])


CH = 25
NCH = NB_W // CH


@functools.partial(
    pl.kernel,
    out_type=jax.ShapeDtypeStruct((NC, N_PAD, F), jnp.float32),
    mesh=_mesh,
    scratch_types=[
        pltpu.VMEM((CH, EB), jnp.int32),
        pltpu.VMEM((CH, EB), jnp.int32),
        pltpu.VMEM((CH, EB), jnp.float32),
        pltpu.VMEM((EB, F), jnp.float32),
        pltpu.VMEM((EB, F), jnp.float32),
        pltpu.VMEM_SHARED((N_PAD, F), jnp.float32),
        pltpu.SemaphoreType.DMA,
        pltpu.SemaphoreType.DMA,
        pltpu.SemaphoreType.DMA,
        pltpu.SemaphoreType.DMA,
    ],
)
def _sc_scatter(fs_hbm, src_hbm, dst_hbm, w_hbm, part_hbm,
                src_v, dst_v, w_v, rows_a, rows_b,
                acc_s, sga, sgb, ssa, ssb):
    c = lax.axis_index("c")
    s = lax.axis_index("s")
    wid = c * NS + s
    lo = s * ROWS_T

    def _zrow(r, _):
        for cc in range(F // 16):
            rows_a[r, pl.ds(cc * 16, 16)] = jnp.zeros((16,), jnp.float32)
        return 0
    lax.fori_loop(0, EB, _zrow, 0)
    for k in range(ROWS_T // EB):
        pltpu.sync_copy(rows_a, acc_s.at[pl.ds(lo + k * EB, EB), :])
    plsc.subcore_barrier()

    def _gather(j, rows, sem):
        pltpu.make_async_copy(fs_hbm.at[src_v.at[j]], rows, sem).start()

    def _gather_wait(j, rows, sem):
        pltpu.make_async_copy(fs_hbm.at[src_v.at[j]], rows, sem).wait()

    def _scatter(j, rows, sem):
        pltpu.make_async_copy(rows, acc_s.at[dst_v.at[j]], sem).start(add=True)

    def _scatter_wait(j, rows, sem):
        pltpu.make_async_copy(rows, acc_s.at[dst_v.at[j]], sem).wait()

    def _scale(j, rows):
        def _grp(g, _):
            sl16 = pl.ds(g * 16, 16)
            wvec = w_v[j, sl16]
            for e in range(16):
                w = wvec[e]
                r = g * 16 + e
                for cc in range(F // 16):
                    sl = pl.ds(cc * 16, 16)
                    rows[r, sl] = rows[r, sl] * w
            return 0
        lax.fori_loop(0, EB // 16, _grp, 0)

    def _chunk(k, _):
        pltpu.sync_copy(src_hbm.at[wid, k], src_v)
        pltpu.sync_copy(dst_hbm.at[wid, k], dst_v)
        pltpu.sync_copy(w_hbm.at[wid, k], w_v)

        _gather(0, rows_a, sga)
        _gather(1, rows_b, sgb)

        def _pair(p, _):
            j0 = 2 * p
            j1 = 2 * p + 1
            _gather_wait(j0, rows_a, sga)
            _scale(j0, rows_a)
            _scatter(j0, rows_a, ssa)
            _gather_wait(j1, rows_b, sgb)
            _scale(j1, rows_b)
            _scatter(j1, rows_b, ssb)

            @pl.when(j0 + 2 < CH)
            def _():
                _scatter_wait(j0, rows_a, ssa)
                _gather(j0 + 2, rows_a, sga)

            @pl.when(j1 + 2 < CH)
            def _():
                _scatter_wait(j1, rows_b, ssb)
                _gather(j1 + 2, rows_b, sgb)
            return 0
        lax.fori_loop(0, CH // 2, _pair, 0)

        jl = CH - 1
        _gather_wait(jl, rows_a, sga)
        _scale(jl, rows_a)
        _scatter(jl, rows_a, ssa)
        _scatter_wait(jl, rows_a, ssa)
        _scatter_wait(jl, rows_b, ssb)
        return 0
    lax.fori_loop(0, NCH, _chunk, 0)
    plsc.subcore_barrier()

    pltpu.sync_copy(acc_s.at[pl.ds(lo, ROWS_T), :],
                    part_hbm.at[c, pl.ds(lo, ROWS_T), :])


def _prescale_body(feat_ref, og0_ref, og1_ref, out_ref):
    d = og0_ref[:, 0] + og1_ref[:, 0]
    ns = lax.rsqrt(jnp.maximum(d, 1.0))
    out_ref[:, :] = feat_ref[:, :] * ns[:, None]


def _tc_prescale(feat, og0, og1):
    blk = 1000
    return pl.pallas_call(
        _prescale_body,
        grid=(N_NODES // blk,),
        in_specs=[
            pl.BlockSpec((blk, F), lambda i: (i, 0)),
            pl.BlockSpec((blk, 1), lambda i: (i, 0)),
            pl.BlockSpec((blk, 1), lambda i: (i, 0)),
        ],
        out_specs=pl.BlockSpec((blk, F), lambda i: (i, 0)),
        out_shape=jax.ShapeDtypeStruct((N_NODES, F), jnp.float32),
    )(feat, og0.reshape(N_PAD, 1), og1.reshape(N_PAD, 1))


def _final_body(p_ref, w_ref, b_ref, ig0_ref, ig1_ref, out_ref):
    acc = p_ref[0, :, :] + p_ref[1, :, :]
    r = jnp.dot(acc, w_ref[:, :], preferred_element_type=jnp.float32)
    d = ig0_ref[:, 0] + ig1_ref[:, 0]
    nd = lax.rsqrt(jnp.maximum(d, 1.0))
    out_ref[:, :] = r * nd[:, None] + b_ref[:, :]


def _tc_final(part, W, b, ig0, ig1):
    blk = 1000
    return pl.pallas_call(
        _final_body,
        grid=(N_NODES // blk,),
        in_specs=[
            pl.BlockSpec((NC, blk, F), lambda i: (0, i, 0)),
            pl.BlockSpec((F, F), lambda i: (0, 0)),
            pl.BlockSpec((1, F), lambda i: (0, 0)),
            pl.BlockSpec((blk, 1), lambda i: (i, 0)),
            pl.BlockSpec((blk, 1), lambda i: (i, 0)),
        ],
        out_specs=pl.BlockSpec((blk, F), lambda i: (i, 0)),
        out_shape=jax.ShapeDtypeStruct((N_NODES, F), jnp.float32),
    )(part, W, b, ig0.reshape(N_PAD, 1), ig1.reshape(N_PAD, 1))


@jax.jit
def kernel(feat, edge_index, edge_weight, W, b):
    src = edge_index[0].astype(jnp.int32)
    dst = edge_index[1].astype(jnp.int32)
    w2 = edge_weight

    og0, ig0, og1, ig1 = _sc_degrees(src.reshape(NW, NB_W, EB),
                                     dst.reshape(NW, NB_W, EB))

    fs = _tc_prescale(feat, og0, og1)
    part = _sc_scatter(fs,
                       src.reshape(NW, NCH, CH, EB),
                       dst.reshape(NW, NCH, CH, EB),
                       w2.reshape(NW, NCH, CH, EB))
    return _tc_final(part, W, b.reshape(1, F), ig0, ig1)


# speedup vs baseline: 10.1222x; 1.0555x over previous
import functools

import jax
import jax.numpy as jnp
from jax import lax
from jax.experimental import pallas as pl
from jax.experimental.pallas import tpu as pltpu
from jax.experimental.pallas import tpu_sc as plsc

N_NODES = 10000
N_EDGES = 320000
F = 128

NC = 2
NS = 16
NW = NC * NS
EB = 80
NB_TOT = N_EDGES // EB
NB_W = NB_TOT // NW
NB_T = NB_TOT // NS
N_PAD = 10240
ROWS_T = N_PAD // NS
CH = 25
NCH = NB_W // CH
NCH_A = NB_T // CH

_mesh = plsc.VectorSubcoreMesh(core_axis_name="c", subcore_axis_name="s")


def _rsqrt16(d):
    i = plsc.bitcast(d, jnp.int32)
    i = jnp.int32(0x5F3759DF) - lax.shift_right_logical(i, 1)
    y = plsc.bitcast(i, jnp.float32)
    for _ in range(3):
        y = y * (1.5 - 0.5 * d * y * y)
    return y


@functools.partial(
    pl.kernel,
    out_type=(
        jax.ShapeDtypeStruct((NC, N_PAD, F), jnp.float32),
        jax.ShapeDtypeStruct((N_PAD,), jnp.float32),
        jax.ShapeDtypeStruct((N_PAD,), jnp.float32),
        jax.ShapeDtypeStruct((N_PAD,), jnp.float32),
        jax.ShapeDtypeStruct((N_PAD,), jnp.float32),
    ),
    mesh=_mesh,
    compiler_params=pltpu.CompilerParams(needs_layout_passes=False),
    scratch_types=[
        pltpu.VMEM((CH, EB), jnp.int32),
        pltpu.VMEM((CH, EB), jnp.int32),
        pltpu.VMEM((CH, EB), jnp.float32),
        pltpu.VMEM((EB, F), jnp.float32),
        pltpu.VMEM((EB, F), jnp.float32),
        pltpu.VMEM((EB, F), jnp.float32),
        pltpu.VMEM((EB,), jnp.float32),
        pltpu.VMEM((EB,), jnp.float32),
        pltpu.VMEM((EB,), jnp.float32),
        pltpu.VMEM((640,), jnp.float32),
        pltpu.VMEM_SHARED((N_PAD, F), jnp.float32),
        pltpu.VMEM_SHARED((N_PAD,), jnp.float32),
        pltpu.VMEM_SHARED((N_PAD,), jnp.float32),
        pltpu.SemaphoreType.DMA,
        pltpu.SemaphoreType.DMA,
        pltpu.SemaphoreType.DMA,
        pltpu.SemaphoreType.DMA,
        pltpu.SemaphoreType.DMA,
        pltpu.SemaphoreType.DMA,
        pltpu.SemaphoreType.DMA,
        pltpu.SemaphoreType.DMA,
        pltpu.SemaphoreType.DMA,
        pltpu.SemaphoreType.DMA,
        pltpu.SemaphoreType.DMA,
    ],
)
def _sc_main(feat_hbm, srca_hbm, src_hbm, dst_hbm, w_hbm,
             part_hbm, nrm0_hbm, nrm1_hbm, ig0_hbm, ig1_hbm,
             src_v, dst_v, w_v, rows_a, rows_b, rows_c, nb_a, nb_b, nb_c,
             buf_v, acc_s, og_s, ig_s,
             sao, sai, sga, sgb, sgc, sna, snb, snc, ssa, ssb, ssc):
    c = lax.axis_index("c")
    s = lax.axis_index("s")
    wid = c * NS + s
    lo = s * ROWS_T

    def _fill(i, _, val):
        buf_v[pl.ds(i * 16, 16)] = jnp.full((16,), val, jnp.float32)
        return 0
    lax.fori_loop(0, 40, functools.partial(_fill, val=0.0), 0)
    pltpu.sync_copy(buf_v, og_s.at[pl.ds(lo, 640)])
    pltpu.sync_copy(buf_v, ig_s.at[pl.ds(lo, 640)])

    def _zrow(r, _):
        for cc in range(F // 16):
            rows_a[r, pl.ds(cc * 16, 16)] = jnp.zeros((16,), jnp.float32)
        return 0
    lax.fori_loop(0, EB, _zrow, 0)
    for k in range(ROWS_T // EB):
        pltpu.sync_copy(rows_a, acc_s.at[pl.ds(lo + k * EB, EB), :])
    plsc.subcore_barrier()

    lax.fori_loop(0, 40, functools.partial(_fill, val=1.0), 0)
    ones_sl = buf_v.at[pl.ds(0, EB)]

    def _chunk_og(k, _):
        pltpu.sync_copy(srca_hbm.at[s, k], src_v)

        def _hist(j, _):
            @pl.when(j > 0)
            def _():
                pltpu.make_async_copy(ones_sl, og_s.at[src_v.at[j]], sao).wait()
            pltpu.make_async_copy(ones_sl, og_s.at[src_v.at[j]], sao).start(add=True)
            return 0
        lax.fori_loop(0, CH, _hist, 0)
        pltpu.make_async_copy(ones_sl, og_s.at[src_v.at[0]], sao).wait()
        return 0
    lax.fori_loop(0, NCH_A, _chunk_og, 0)

    def _chunk_ig(k, _):
        pltpu.sync_copy(dst_hbm.at[wid, k], dst_v)

        def _hist(j, _):
            @pl.when(j > 0)
            def _():
                pltpu.make_async_copy(ones_sl, ig_s.at[dst_v.at[j]], sai).wait()
            pltpu.make_async_copy(ones_sl, ig_s.at[dst_v.at[j]], sai).start(add=True)
            return 0
        lax.fori_loop(0, CH, _hist, 0)
        pltpu.make_async_copy(ones_sl, ig_s.at[dst_v.at[0]], sai).wait()
        return 0
    lax.fori_loop(0, NCH, _chunk_ig, 0)
    plsc.subcore_barrier()

    pltpu.sync_copy(og_s.at[pl.ds(lo, 640)], buf_v)

    def _norm(i, _):
        sl = pl.ds(i * 16, 16)
        buf_v[sl] = _rsqrt16(jnp.maximum(buf_v[sl], 1.0))
        return 0
    lax.fori_loop(0, 40, _norm, 0)

    @pl.when(c == 0)
    def _():
        pltpu.sync_copy(buf_v, nrm0_hbm.at[pl.ds(lo, 640)])
        pltpu.sync_copy(ig_s.at[pl.ds(lo, 640)], ig0_hbm.at[pl.ds(lo, 640)])

    @pl.when(c == 1)
    def _():
        pltpu.sync_copy(buf_v, nrm1_hbm.at[pl.ds(lo, 640)])
        pltpu.sync_copy(ig_s.at[pl.ds(lo, 640)], ig1_hbm.at[pl.ds(lo, 640)])
    plsc.subcore_barrier()

    def _gather(j, rows, sem):
        pltpu.make_async_copy(feat_hbm.at[src_v.at[j]], rows, sem).start()

    def _gather_wait(j, rows, sem):
        pltpu.make_async_copy(feat_hbm.at[src_v.at[j]], rows, sem).wait()

    def _ngather(j, nb, sem):
        @pl.when(c == 0)
        def _():
            pltpu.make_async_copy(nrm0_hbm.at[src_v.at[j]], nb, sem).start()

        @pl.when(c == 1)
        def _():
            pltpu.make_async_copy(nrm1_hbm.at[src_v.at[j]], nb, sem).start()

    def _ngather_wait(j, nb, sem):
        pltpu.make_async_copy(nrm0_hbm.at[src_v.at[j]], nb, sem).wait()

    def _scatter(j, rows, sem):
        pltpu.make_async_copy(rows, acc_s.at[dst_v.at[j]], sem).start(add=True)

    def _scatter_wait(j, rows, sem):
        pltpu.make_async_copy(rows, acc_s.at[dst_v.at[j]], sem).wait()

    def _scale(j, rows, nb):
        def _grp(g, _):
            sl16 = pl.ds(g * 16, 16)
            wvec = w_v[j, sl16] * nb[sl16]
            for e in range(16):
                w = wvec[e]
                r = g * 16 + e
                for cc in range(F // 16):
                    sl = pl.ds(cc * 16, 16)
                    rows[r, sl] = rows[r, sl] * w
            return 0
        lax.fori_loop(0, EB // 16, _grp, 0)

    slots = ((rows_a, nb_a, sga, sna, ssa),
             (rows_b, nb_b, sgb, snb, ssb),
             (rows_c, nb_c, sgc, snc, ssc))

    def _chunk(k, _):
        pltpu.sync_copy(src_hbm.at[wid, k], src_v)
        pltpu.sync_copy(dst_hbm.at[wid, k], dst_v)
        pltpu.sync_copy(w_hbm.at[wid, k], w_v)

        for jj, (rows, nb, sg, sn, ss) in enumerate(slots):
            _gather(jj, rows, sg)
            _ngather(jj, nb, sn)

        def _triple(t, _):
            for i in range(3):
                rows, nb, sg, sn, ss = slots[i]
                j = 3 * t + i
                _gather_wait(j, rows, sg)
                _ngather_wait(j, nb, sn)
                _scale(j, rows, nb)
                _scatter(j, rows, ss)
                rows_p, nb_p, sg_p, sn_p, ss_p = slots[(i + 2) % 3]
                jp = j - 1

                @pl.when((jp >= 0) & (jp + 3 <= CH - 1))
                def _():
                    _scatter_wait(jp, rows_p, ss_p)
                    _gather(jp + 3, rows_p, sg_p)
                    _ngather(jp + 3, nb_p, sn_p)
            return 0
        lax.fori_loop(0, CH // 3, _triple, 0)

        jl = CH - 1
        _gather_wait(jl, rows_a, sga)
        _ngather_wait(jl, nb_a, sna)
        _scale(jl, rows_a, nb_a)
        _scatter(jl, rows_a, ssa)
        _scatter_wait(CH - 3, rows_b, ssb)
        _scatter_wait(CH - 2, rows_c, ssc)
        _scatter_wait(jl, rows_a, ssa)
        return 0
    lax.fori_loop(0, NCH, _chunk, 0)
    plsc.subcore_barrier()

    pltpu.sync_copy(acc_s.at[pl.ds(lo, ROWS_T), :],
                    part_hbm.at[c, pl.ds(lo, ROWS_T), :])


def _final_body(p_ref, w_ref, b_ref, ig0_ref, ig1_ref, out_ref):
    acc = p_ref[0, :, :] + p_ref[1, :, :]
    r = jnp.dot(acc, w_ref[:, :], preferred_element_type=jnp.float32)
    d = ig0_ref[:, 0] + ig1_ref[:, 0]
    nd = lax.rsqrt(jnp.maximum(d, 1.0))
    out_ref[:, :] = r * nd[:, None] + b_ref[:, :]


def _tc_final(part, W, b, ig0, ig1):
    blk = 1000
    return pl.pallas_call(
        _final_body,
        grid=(N_NODES // blk,),
        in_specs=[
            pl.BlockSpec((NC, blk, F), lambda i: (0, i, 0)),
            pl.BlockSpec((F, F), lambda i: (0, 0)),
            pl.BlockSpec((1, F), lambda i: (0, 0)),
            pl.BlockSpec((blk, 1), lambda i: (i, 0)),
            pl.BlockSpec((blk, 1), lambda i: (i, 0)),
        ],
        out_specs=pl.BlockSpec((blk, F), lambda i: (i, 0)),
        out_shape=jax.ShapeDtypeStruct((N_NODES, F), jnp.float32),
    )(part, W, b, ig0.reshape(N_PAD, 1), ig1.reshape(N_PAD, 1))


@jax.jit
def kernel(feat, edge_index, edge_weight, W, b):
    src = edge_index[0].astype(jnp.int32)
    dst = edge_index[1].astype(jnp.int32)

    part, _, _, ig0, ig1 = _sc_main(
        feat,
        src.reshape(NS, NCH_A, CH, EB),
        src.reshape(NW, NCH, CH, EB),
        dst.reshape(NW, NCH, CH, EB),
        edge_weight.reshape(NW, NCH, CH, EB),
    )
    return _tc_final(part, W, b.reshape(1, F), ig0, ig1)


# speedup vs baseline: 10.9172x; 1.0785x over previous
import functools

import jax
import jax.numpy as jnp
from jax import lax
from jax.experimental import pallas as pl
from jax.experimental.pallas import tpu as pltpu
from jax.experimental.pallas import tpu_sc as plsc

N_NODES = 10000
N_EDGES = 320000
F = 128

NC = 2
NS = 16
NW = NC * NS
EB = 80
NB_TOT = N_EDGES // EB
NB_W = NB_TOT // NW
N_PAD = 10240
ROWS_T = N_PAD // NS

_mesh = plsc.VectorSubcoreMesh(core_axis_name="c", subcore_axis_name="s")


@functools.partial(
    pl.kernel,
    out_type=(
        jax.ShapeDtypeStruct((N_PAD,), jnp.float32),
        jax.ShapeDtypeStruct((N_PAD,), jnp.float32),
    ),
    mesh=_mesh,
    scratch_types=[
        pltpu.VMEM((NB_W, EB), jnp.int32),
        pltpu.VMEM((640,), jnp.float32),
        pltpu.VMEM_SHARED((N_PAD,), jnp.float32),
        pltpu.SemaphoreType.DMA,
    ],
)
def _sc_degrees(src_hbm, og0_hbm, og1_hbm, src_v, buf_v, og_s, sem_o):
    c = lax.axis_index("c")
    s = lax.axis_index("s")
    wid = c * NS + s

    pltpu.sync_copy(src_hbm.at[wid], src_v)

    def _fill(i, _, val):
        buf_v[pl.ds(i * 16, 16)] = jnp.full((16,), val, jnp.float32)
        return 0
    lax.fori_loop(0, 40, functools.partial(_fill, val=0.0), 0)
    lo = s * 640
    pltpu.sync_copy(buf_v, og_s.at[pl.ds(lo, 640)])
    plsc.subcore_barrier()

    lax.fori_loop(0, 40, functools.partial(_fill, val=1.0), 0)

    ones_sl = buf_v.at[pl.ds(0, EB)]

    def _hist(j, _):
        @pl.when(j > 0)
        def _():
            pltpu.make_async_copy(ones_sl, og_s.at[src_v.at[j]], sem_o).wait()
        pltpu.make_async_copy(ones_sl, og_s.at[src_v.at[j]], sem_o).start(add=True)
        return 0
    lax.fori_loop(0, NB_W, _hist, 0)
    pltpu.make_async_copy(ones_sl, og_s.at[src_v.at[0]], sem_o).wait()
    plsc.subcore_barrier()

    @pl.when(c == 0)
    def _():
        pltpu.sync_copy(og_s.at[pl.ds(lo, 640)], og0_hbm.at[pl.ds(lo, 640)])

    @pl.when(c == 1)
    def _():
        pltpu.sync_copy(og_s.at[pl.ds(lo, 640)], og1_hbm.at[pl.ds(lo, 640)])


CH = 25
NCH = NB_W // CH


@functools.partial(
    pl.kernel,
    out_type=(
        jax.ShapeDtypeStruct((NC, N_PAD, F), jnp.float32),
        jax.ShapeDtypeStruct((N_PAD,), jnp.float32),
        jax.ShapeDtypeStruct((N_PAD,), jnp.float32),
    ),
    mesh=_mesh,
    scratch_types=[
        pltpu.VMEM((CH, EB), jnp.int32),
        pltpu.VMEM((CH, EB), jnp.int32),
        pltpu.VMEM((CH, EB), jnp.float32),
        pltpu.VMEM((EB, F), jnp.float32),
        pltpu.VMEM((EB, F), jnp.float32),
        pltpu.VMEM((EB, F), jnp.float32),
        pltpu.VMEM((640,), jnp.float32),
        pltpu.VMEM_SHARED((N_PAD, F), jnp.float32),
        pltpu.VMEM_SHARED((N_PAD,), jnp.float32),
        pltpu.SemaphoreType.DMA,
        pltpu.SemaphoreType.DMA,
        pltpu.SemaphoreType.DMA,
        pltpu.SemaphoreType.DMA,
        pltpu.SemaphoreType.DMA,
        pltpu.SemaphoreType.DMA,
        pltpu.SemaphoreType.DMA,
    ],
)
def _sc_scatter(fs_hbm, src_hbm, dst_hbm, w_hbm, part_hbm, ig0_hbm, ig1_hbm,
                src_v, dst_v, w_v, rows_a, rows_b, rows_c, buf_v,
                acc_s, ig_s, sga, sgb, sgc, ssa, ssb, ssc, sai):
    c = lax.axis_index("c")
    s = lax.axis_index("s")
    wid = c * NS + s
    lo = s * ROWS_T

    def _fill(i, _, val):
        buf_v[pl.ds(i * 16, 16)] = jnp.full((16,), val, jnp.float32)
        return 0
    lax.fori_loop(0, 40, functools.partial(_fill, val=0.0), 0)
    pltpu.sync_copy(buf_v, ig_s.at[pl.ds(lo, 640)])

    def _zrow(r, _):
        for cc in range(F // 16):
            rows_a[r, pl.ds(cc * 16, 16)] = jnp.zeros((16,), jnp.float32)
        return 0
    lax.fori_loop(0, EB, _zrow, 0)
    for k in range(ROWS_T // EB):
        pltpu.sync_copy(rows_a, acc_s.at[pl.ds(lo + k * EB, EB), :])
    lax.fori_loop(0, 40, functools.partial(_fill, val=1.0), 0)
    ones_sl = buf_v.at[pl.ds(0, EB)]
    plsc.subcore_barrier()

    def _gather(j, rows, sem):
        pltpu.make_async_copy(fs_hbm.at[src_v.at[j]], rows, sem).start()

    def _gather_wait(j, rows, sem):
        pltpu.make_async_copy(fs_hbm.at[src_v.at[j]], rows, sem).wait()

    def _scatter(j, rows, sem):
        pltpu.make_async_copy(rows, acc_s.at[dst_v.at[j]], sem).start(add=True)

    def _scatter_wait(j, rows, sem):
        pltpu.make_async_copy(rows, acc_s.at[dst_v.at[j]], sem).wait()

    def _ig_add(j, _):
        @pl.when(j > 0)
        def _():
            pltpu.make_async_copy(ones_sl, ig_s.at[dst_v.at[j]], sai).wait()
        pltpu.make_async_copy(ones_sl, ig_s.at[dst_v.at[j]], sai).start(add=True)

    def _scale(j, rows):
        def _grp(g, _):
            sl16 = pl.ds(g * 16, 16)
            wvec = w_v[j, sl16]
            for e in range(16):
                w = wvec[e]
                r = g * 16 + e
                for cc in range(F // 16):
                    sl = pl.ds(cc * 16, 16)
                    rows[r, sl] = rows[r, sl] * w
            return 0
        lax.fori_loop(0, EB // 16, _grp, 0)

    slots = ((rows_a, sga, ssa), (rows_b, sgb, ssb), (rows_c, sgc, ssc))

    def _chunk(k, _):
        pltpu.sync_copy(src_hbm.at[wid, k], src_v)
        pltpu.sync_copy(dst_hbm.at[wid, k], dst_v)
        pltpu.sync_copy(w_hbm.at[wid, k], w_v)

        _gather(0, rows_a, sga)
        _gather(1, rows_b, sgb)
        _gather(2, rows_c, sgc)

        def _triple(t, _):
            for i in range(3):
                rows, sg, ss = slots[i]
                j = 3 * t + i
                _gather_wait(j, rows, sg)
                _scale(j, rows)
                _scatter(j, rows, ss)
                _ig_add(j, None)
                rows_p, sg_p, ss_p = slots[(i + 2) % 3]
                jp = j - 1

                @pl.when((jp >= 0) & (jp + 3 <= CH - 1))
                def _():
                    _scatter_wait(jp, rows_p, ss_p)
                    _gather(jp + 3, rows_p, sg_p)
            return 0
        lax.fori_loop(0, CH // 3, _triple, 0)

        jl = CH - 1
        _gather_wait(jl, rows_a, sga)
        _scale(jl, rows_a)
        _scatter(jl, rows_a, ssa)
        _ig_add(jl, None)
        _scatter_wait(CH - 3, rows_b, ssb)
        _scatter_wait(CH - 2, rows_c, ssc)
        _scatter_wait(jl, rows_a, ssa)
        pltpu.make_async_copy(ones_sl, ig_s.at[dst_v.at[0]], sai).wait()
        return 0
    lax.fori_loop(0, NCH, _chunk, 0)
    plsc.subcore_barrier()

    pltpu.sync_copy(acc_s.at[pl.ds(lo, ROWS_T), :],
                    part_hbm.at[c, pl.ds(lo, ROWS_T), :])

    @pl.when(c == 0)
    def _():
        pltpu.sync_copy(ig_s.at[pl.ds(lo, 640)], ig0_hbm.at[pl.ds(lo, 640)])

    @pl.when(c == 1)
    def _():
        pltpu.sync_copy(ig_s.at[pl.ds(lo, 640)], ig1_hbm.at[pl.ds(lo, 640)])


def _prescale_body(feat_ref, og0_ref, og1_ref, out_ref):
    d = og0_ref[:, 0] + og1_ref[:, 0]
    ns = lax.rsqrt(jnp.maximum(d, 1.0))
    out_ref[:, :] = feat_ref[:, :] * ns[:, None]


def _tc_prescale(feat, og0, og1):
    blk = 1000
    return pl.pallas_call(
        _prescale_body,
        grid=(N_NODES // blk,),
        in_specs=[
            pl.BlockSpec((blk, F), lambda i: (i, 0)),
            pl.BlockSpec((blk, 1), lambda i: (i, 0)),
            pl.BlockSpec((blk, 1), lambda i: (i, 0)),
        ],
        out_specs=pl.BlockSpec((blk, F), lambda i: (i, 0)),
        out_shape=jax.ShapeDtypeStruct((N_NODES, F), jnp.float32),
    )(feat, og0.reshape(N_PAD, 1), og1.reshape(N_PAD, 1))


def _final_body(p_ref, w_ref, b_ref, ig0_ref, ig1_ref, out_ref):
    acc = p_ref[0, :, :] + p_ref[1, :, :]
    r = jnp.dot(acc, w_ref[:, :], preferred_element_type=jnp.float32)
    d = ig0_ref[:, 0] + ig1_ref[:, 0]
    nd = lax.rsqrt(jnp.maximum(d, 1.0))
    out_ref[:, :] = r * nd[:, None] + b_ref[:, :]


def _tc_final(part, W, b, ig0, ig1):
    blk = 1000
    return pl.pallas_call(
        _final_body,
        grid=(N_NODES // blk,),
        in_specs=[
            pl.BlockSpec((NC, blk, F), lambda i: (0, i, 0)),
            pl.BlockSpec((F, F), lambda i: (0, 0)),
            pl.BlockSpec((1, F), lambda i: (0, 0)),
            pl.BlockSpec((blk, 1), lambda i: (i, 0)),
            pl.BlockSpec((blk, 1), lambda i: (i, 0)),
        ],
        out_specs=pl.BlockSpec((blk, F), lambda i: (i, 0)),
        out_shape=jax.ShapeDtypeStruct((N_NODES, F), jnp.float32),
    )(part, W, b, ig0.reshape(N_PAD, 1), ig1.reshape(N_PAD, 1))


@jax.jit
def kernel(feat, edge_index, edge_weight, W, b):
    src = edge_index[0].astype(jnp.int32)
    dst = edge_index[1].astype(jnp.int32)
    w2 = edge_weight

    og0, og1 = _sc_degrees(src.reshape(NW, NB_W, EB))

    fs = _tc_prescale(feat, og0, og1)
    part, ig0, ig1 = _sc_scatter(fs,
                                 src.reshape(NW, NCH, CH, EB),
                                 dst.reshape(NW, NCH, CH, EB),
                                 w2.reshape(NW, NCH, CH, EB))
    return _tc_final(part, W, b.reshape(1, F), ig0, ig1)


# speedup vs baseline: 11.1261x; 1.0191x over previous
import functools

import jax
import jax.numpy as jnp
from jax import lax
from jax.experimental import pallas as pl
from jax.experimental.pallas import tpu as pltpu
from jax.experimental.pallas import tpu_sc as plsc

N_NODES = 10000
N_EDGES = 320000
F = 128

NC = 2
NS = 16
NW = NC * NS
EB = 80
NB_TOT = N_EDGES // EB
NB_W = NB_TOT // NW
N_PAD = 10240
ROWS_T = N_PAD // NS

_mesh = plsc.VectorSubcoreMesh(core_axis_name="c", subcore_axis_name="s")


@functools.partial(
    pl.kernel,
    out_type=(
        jax.ShapeDtypeStruct((N_PAD,), jnp.float32),
        jax.ShapeDtypeStruct((N_PAD,), jnp.float32),
    ),
    mesh=_mesh,
    scratch_types=[
        pltpu.VMEM((NB_W, EB), jnp.int32),
        pltpu.VMEM((640,), jnp.float32),
        pltpu.VMEM_SHARED((N_PAD,), jnp.float32),
        pltpu.SemaphoreType.DMA,
    ],
)
def _sc_degrees(src_hbm, og0_hbm, og1_hbm, src_v, buf_v, og_s, sem_o):
    c = lax.axis_index("c")
    s = lax.axis_index("s")
    wid = c * NS + s

    pltpu.sync_copy(src_hbm.at[wid], src_v)

    def _fill(i, _, val):
        buf_v[pl.ds(i * 16, 16)] = jnp.full((16,), val, jnp.float32)
        return 0
    lax.fori_loop(0, 40, functools.partial(_fill, val=0.0), 0)
    lo = s * 640
    pltpu.sync_copy(buf_v, og_s.at[pl.ds(lo, 640)])
    plsc.subcore_barrier()

    lax.fori_loop(0, 40, functools.partial(_fill, val=1.0), 0)

    ones_sl = buf_v.at[pl.ds(0, EB)]

    def _hist(j, _):
        @pl.when(j > 0)
        def _():
            pltpu.make_async_copy(ones_sl, og_s.at[src_v.at[j]], sem_o).wait()
        pltpu.make_async_copy(ones_sl, og_s.at[src_v.at[j]], sem_o).start(add=True)
        return 0
    lax.fori_loop(0, NB_W, _hist, 0)
    pltpu.make_async_copy(ones_sl, og_s.at[src_v.at[0]], sem_o).wait()
    plsc.subcore_barrier()

    @pl.when(c == 0)
    def _():
        pltpu.sync_copy(og_s.at[pl.ds(lo, 640)], og0_hbm.at[pl.ds(lo, 640)])

    @pl.when(c == 1)
    def _():
        pltpu.sync_copy(og_s.at[pl.ds(lo, 640)], og1_hbm.at[pl.ds(lo, 640)])


CH = 25
NCH = NB_W // CH


@functools.partial(
    pl.kernel,
    out_type=(
        jax.ShapeDtypeStruct((NC, N_PAD, F), jnp.float32),
        jax.ShapeDtypeStruct((N_PAD,), jnp.float32),
        jax.ShapeDtypeStruct((N_PAD,), jnp.float32),
    ),
    mesh=_mesh,
    scratch_types=[
        pltpu.VMEM((CH, EB), jnp.int32),
        pltpu.VMEM((CH, EB), jnp.int32),
        pltpu.VMEM((CH, EB), jnp.float32),
        pltpu.VMEM((EB, F), jnp.float32),
        pltpu.VMEM((EB, F), jnp.float32),
        pltpu.VMEM((EB, F), jnp.float32),
        pltpu.VMEM((640,), jnp.float32),
        pltpu.VMEM_SHARED((N_PAD, F), jnp.float32),
        pltpu.VMEM_SHARED((N_PAD,), jnp.float32),
        pltpu.SemaphoreType.DMA,
        pltpu.SemaphoreType.DMA,
        pltpu.SemaphoreType.DMA,
        pltpu.SemaphoreType.DMA,
        pltpu.SemaphoreType.DMA,
        pltpu.SemaphoreType.DMA,
        pltpu.SemaphoreType.DMA,
    ],
)
def _sc_scatter(fs_hbm, src_hbm, dst_hbm, w_hbm, part_hbm, ig0_hbm, ig1_hbm,
                src_v, dst_v, w_v, rows_a, rows_b, rows_c, buf_v,
                acc_s, ig_s, sga, sgb, sgc, ssa, ssb, ssc, sai):
    c = lax.axis_index("c")
    s = lax.axis_index("s")
    wid = c * NS + s
    lo = s * ROWS_T

    def _fill(i, _, val):
        buf_v[pl.ds(i * 16, 16)] = jnp.full((16,), val, jnp.float32)
        return 0
    lax.fori_loop(0, 40, functools.partial(_fill, val=0.0), 0)
    pltpu.sync_copy(buf_v, ig_s.at[pl.ds(lo, 640)])

    def _zrow(r, _):
        for cc in range(F // 16):
            rows_a[r, pl.ds(cc * 16, 16)] = jnp.zeros((16,), jnp.float32)
        return 0
    lax.fori_loop(0, EB, _zrow, 0)
    for k in range(ROWS_T // EB):
        pltpu.sync_copy(rows_a, acc_s.at[pl.ds(lo + k * EB, EB), :])
    lax.fori_loop(0, 40, functools.partial(_fill, val=1.0), 0)
    ones_sl = buf_v.at[pl.ds(0, EB)]
    plsc.subcore_barrier()

    def _gather(j, rows, sem):
        pltpu.make_async_copy(fs_hbm.at[src_v.at[j]], rows, sem).start()

    def _gather_wait(j, rows, sem):
        pltpu.make_async_copy(fs_hbm.at[src_v.at[j]], rows, sem).wait()

    def _scatter(j, rows, sem):
        pltpu.make_async_copy(rows, acc_s.at[dst_v.at[j]], sem).start(add=True)

    def _scatter_wait(j, rows, sem):
        pltpu.make_async_copy(rows, acc_s.at[dst_v.at[j]], sem).wait()

    def _ig_add(j, _):
        @pl.when(j > 0)
        def _():
            pltpu.make_async_copy(ones_sl, ig_s.at[dst_v.at[j]], sai).wait()
        pltpu.make_async_copy(ones_sl, ig_s.at[dst_v.at[j]], sai).start(add=True)

    def _scale(j, rows):
        def _grp(g, _):
            sl16 = pl.ds(g * 16, 16)
            wvec = w_v[j, sl16]
            for e in range(16):
                w = wvec[e]
                r = g * 16 + e
                for cc in range(F // 16):
                    sl = pl.ds(cc * 16, 16)
                    rows[r, sl] = rows[r, sl] * w
            return 0
        lax.fori_loop(0, EB // 16, _grp, 0)

    slots = ((rows_a, sga, ssa), (rows_b, sgb, ssb), (rows_c, sgc, ssc))

    def _chunk(k, _):
        pltpu.sync_copy(src_hbm.at[wid, k], src_v)
        pltpu.sync_copy(dst_hbm.at[wid, k], dst_v)
        pltpu.sync_copy(w_hbm.at[wid, k], w_v)

        _gather(0, rows_a, sga)
        _gather(1, rows_b, sgb)
        _gather(2, rows_c, sgc)

        def _triple(t, _):
            for i in range(3):
                rows, sg, ss = slots[i]
                j = 3 * t + i
                _gather_wait(j, rows, sg)
                _scale(j, rows)
                _scatter(j, rows, ss)
                _ig_add(j, None)
                rows_p, sg_p, ss_p = slots[(i + 2) % 3]
                jp = j - 1

                @pl.when((jp >= 0) & (jp + 3 <= CH - 1))
                def _():
                    _scatter_wait(jp, rows_p, ss_p)
                    _gather(jp + 3, rows_p, sg_p)
            return 0
        lax.fori_loop(0, CH // 3, _triple, 0)

        jl = CH - 1
        _gather_wait(jl, rows_a, sga)
        _scale(jl, rows_a)
        _scatter(jl, rows_a, ssa)
        _ig_add(jl, None)
        _scatter_wait(CH - 3, rows_b, ssb)
        _scatter_wait(CH - 2, rows_c, ssc)
        _scatter_wait(jl, rows_a, ssa)
        pltpu.make_async_copy(ones_sl, ig_s.at[dst_v.at[0]], sai).wait()
        return 0
    lax.fori_loop(0, NCH, _chunk, 0)
    plsc.subcore_barrier()

    pltpu.sync_copy(acc_s.at[pl.ds(lo, ROWS_T), :],
                    part_hbm.at[c, pl.ds(lo, ROWS_T), :])

    @pl.when(c == 0)
    def _():
        pltpu.sync_copy(ig_s.at[pl.ds(lo, 640)], ig0_hbm.at[pl.ds(lo, 640)])

    @pl.when(c == 1)
    def _():
        pltpu.sync_copy(ig_s.at[pl.ds(lo, 640)], ig1_hbm.at[pl.ds(lo, 640)])


def _prescale_body(feat_ref, og0_ref, og1_ref, out_ref):
    d = og0_ref[:, 0] + og1_ref[:, 0]
    ns = lax.rsqrt(jnp.maximum(d, 1.0))
    out_ref[:, :] = feat_ref[:, :] * ns[:, None]


def _tc_prescale(feat, og0, og1):
    blk = 2000
    return pl.pallas_call(
        _prescale_body,
        grid=(N_NODES // blk,),
        in_specs=[
            pl.BlockSpec((blk, F), lambda i: (i, 0)),
            pl.BlockSpec((blk, 1), lambda i: (i, 0)),
            pl.BlockSpec((blk, 1), lambda i: (i, 0)),
        ],
        out_specs=pl.BlockSpec((blk, F), lambda i: (i, 0)),
        out_shape=jax.ShapeDtypeStruct((N_NODES, F), jnp.float32),
    )(feat, og0.reshape(N_PAD, 1), og1.reshape(N_PAD, 1))


def _final_body(p_ref, w_ref, b_ref, ig0_ref, ig1_ref, out_ref):
    acc = p_ref[0, :, :] + p_ref[1, :, :]
    r = jnp.dot(acc, w_ref[:, :], preferred_element_type=jnp.float32)
    d = ig0_ref[:, 0] + ig1_ref[:, 0]
    nd = lax.rsqrt(jnp.maximum(d, 1.0))
    out_ref[:, :] = r * nd[:, None] + b_ref[:, :]


def _tc_final(part, W, b, ig0, ig1):
    blk = 2000
    return pl.pallas_call(
        _final_body,
        grid=(N_NODES // blk,),
        in_specs=[
            pl.BlockSpec((NC, blk, F), lambda i: (0, i, 0)),
            pl.BlockSpec((F, F), lambda i: (0, 0)),
            pl.BlockSpec((1, F), lambda i: (0, 0)),
            pl.BlockSpec((blk, 1), lambda i: (i, 0)),
            pl.BlockSpec((blk, 1), lambda i: (i, 0)),
        ],
        out_specs=pl.BlockSpec((blk, F), lambda i: (i, 0)),
        out_shape=jax.ShapeDtypeStruct((N_NODES, F), jnp.float32),
    )(part, W, b, ig0.reshape(N_PAD, 1), ig1.reshape(N_PAD, 1))


@jax.jit
def kernel(feat, edge_index, edge_weight, W, b):
    src = edge_index[0].astype(jnp.int32)
    dst = edge_index[1].astype(jnp.int32)
    w2 = edge_weight

    og0, og1 = _sc_degrees(src.reshape(NW, NB_W, EB))

    fs = _tc_prescale(feat, og0, og1)
    part, ig0, ig1 = _sc_scatter(fs,
                                 src.reshape(NW, NCH, CH, EB),
                                 dst.reshape(NW, NCH, CH, EB),
                                 w2.reshape(NW, NCH, CH, EB))
    return _tc_final(part, W, b.reshape(1, F), ig0, ig1)


# speedup vs baseline: 11.2131x; 1.0078x over previous
import functools

import jax
import jax.numpy as jnp
from jax import lax
from jax.experimental import pallas as pl
from jax.experimental.pallas import tpu as pltpu
from jax.experimental.pallas import tpu_sc as plsc

N_NODES = 10000
N_EDGES = 320000
F = 128

NC = 2
NS = 16
NW = NC * NS
EB = 80
NB_TOT = N_EDGES // EB
NB_W = NB_TOT // NW
N_PAD = 10240
ROWS_T = N_PAD // NS

_mesh = plsc.VectorSubcoreMesh(core_axis_name="c", subcore_axis_name="s")


@functools.partial(
    pl.kernel,
    out_type=(
        jax.ShapeDtypeStruct((N_PAD,), jnp.float32),
        jax.ShapeDtypeStruct((N_PAD,), jnp.float32),
    ),
    mesh=_mesh,
    scratch_types=[
        pltpu.VMEM((NB_W, EB), jnp.int32),
        pltpu.VMEM((640,), jnp.float32),
        pltpu.VMEM_SHARED((N_PAD,), jnp.float32),
        pltpu.SemaphoreType.DMA,
    ],
)
def _sc_degrees(src_hbm, og0_hbm, og1_hbm, src_v, buf_v, og_s, sem_o):
    c = lax.axis_index("c")
    s = lax.axis_index("s")
    wid = c * NS + s

    pltpu.sync_copy(src_hbm.at[wid], src_v)

    def _fill(i, _, val):
        buf_v[pl.ds(i * 16, 16)] = jnp.full((16,), val, jnp.float32)
        return 0
    lax.fori_loop(0, 40, functools.partial(_fill, val=0.0), 0)
    lo = s * 640
    pltpu.sync_copy(buf_v, og_s.at[pl.ds(lo, 640)])
    plsc.subcore_barrier()

    lax.fori_loop(0, 40, functools.partial(_fill, val=1.0), 0)

    ones_sl = buf_v.at[pl.ds(0, EB)]

    def _hist(j, _):
        @pl.when(j > 0)
        def _():
            pltpu.make_async_copy(ones_sl, og_s.at[src_v.at[j]], sem_o).wait()
        pltpu.make_async_copy(ones_sl, og_s.at[src_v.at[j]], sem_o).start(add=True)
        return 0
    lax.fori_loop(0, NB_W, _hist, 0)
    pltpu.make_async_copy(ones_sl, og_s.at[src_v.at[0]], sem_o).wait()
    plsc.subcore_barrier()

    @pl.when(c == 0)
    def _():
        pltpu.sync_copy(og_s.at[pl.ds(lo, 640)], og0_hbm.at[pl.ds(lo, 640)])

    @pl.when(c == 1)
    def _():
        pltpu.sync_copy(og_s.at[pl.ds(lo, 640)], og1_hbm.at[pl.ds(lo, 640)])


CH = 25
NCH = NB_W // CH


@functools.partial(
    pl.kernel,
    out_type=(
        jax.ShapeDtypeStruct((NC, N_PAD, F), jnp.float32),
        jax.ShapeDtypeStruct((N_PAD,), jnp.float32),
        jax.ShapeDtypeStruct((N_PAD,), jnp.float32),
    ),
    mesh=_mesh,
    scratch_types=[
        pltpu.VMEM((CH, EB), jnp.int32),
        pltpu.VMEM((CH, EB), jnp.int32),
        pltpu.VMEM((CH, EB), jnp.float32),
        pltpu.VMEM((EB, F), jnp.float32),
        pltpu.VMEM((EB, F), jnp.float32),
        pltpu.VMEM((EB, F), jnp.float32),
        pltpu.VMEM((640,), jnp.float32),
        pltpu.VMEM_SHARED((N_PAD, F), jnp.float32),
        pltpu.VMEM_SHARED((N_PAD,), jnp.float32),
        pltpu.SemaphoreType.DMA,
        pltpu.SemaphoreType.DMA,
        pltpu.SemaphoreType.DMA,
        pltpu.SemaphoreType.DMA,
        pltpu.SemaphoreType.DMA,
        pltpu.SemaphoreType.DMA,
        pltpu.SemaphoreType.DMA,
    ],
)
def _sc_scatter(fs_hbm, src_hbm, dst_hbm, w_hbm, part_hbm, ig0_hbm, ig1_hbm,
                src_v, dst_v, w_v, rows_a, rows_b, rows_c, buf_v,
                acc_s, ig_s, sga, sgb, sgc, ssa, ssb, ssc, sai):
    c = lax.axis_index("c")
    s = lax.axis_index("s")
    wid = c * NS + s
    lo = s * ROWS_T

    def _fill(i, _, val):
        buf_v[pl.ds(i * 16, 16)] = jnp.full((16,), val, jnp.float32)
        return 0
    lax.fori_loop(0, 40, functools.partial(_fill, val=0.0), 0)
    pltpu.sync_copy(buf_v, ig_s.at[pl.ds(lo, 640)])

    def _zrow(r, _):
        for cc in range(F // 16):
            rows_a[r, pl.ds(cc * 16, 16)] = jnp.zeros((16,), jnp.float32)
        return 0
    lax.fori_loop(0, EB, _zrow, 0)
    for k in range(ROWS_T // EB):
        pltpu.sync_copy(rows_a, acc_s.at[pl.ds(lo + k * EB, EB), :])
    lax.fori_loop(0, 40, functools.partial(_fill, val=1.0), 0)
    ones_sl = buf_v.at[pl.ds(0, EB)]
    plsc.subcore_barrier()

    def _gather(j, rows, sem):
        pltpu.make_async_copy(fs_hbm.at[src_v.at[j]], rows, sem).start()

    def _gather_wait(j, rows, sem):
        pltpu.make_async_copy(fs_hbm.at[src_v.at[j]], rows, sem).wait()

    def _scatter(j, rows, sem):
        pltpu.make_async_copy(rows, acc_s.at[dst_v.at[j]], sem).start(add=True)

    def _scatter_wait(j, rows, sem):
        pltpu.make_async_copy(rows, acc_s.at[dst_v.at[j]], sem).wait()

    def _ig_add(j, _):
        @pl.when(j > 0)
        def _():
            pltpu.make_async_copy(ones_sl, ig_s.at[dst_v.at[j]], sai).wait()
        pltpu.make_async_copy(ones_sl, ig_s.at[dst_v.at[j]], sai).start(add=True)

    def _scale(j, rows):
        def _grp(g, _):
            sl16 = pl.ds(g * 16, 16)
            wvec = w_v[j, sl16]
            for e in range(16):
                w = wvec[e]
                r = g * 16 + e
                for cc in range(F // 16):
                    sl = pl.ds(cc * 16, 16)
                    rows[r, sl] = rows[r, sl] * w
            return 0
        lax.fori_loop(0, EB // 16, _grp, 0)

    slots = ((rows_a, sga, ssa), (rows_b, sgb, ssb), (rows_c, sgc, ssc))

    def _chunk(k, _):
        pltpu.sync_copy(src_hbm.at[wid, k], src_v)
        pltpu.sync_copy(dst_hbm.at[wid, k], dst_v)
        pltpu.sync_copy(w_hbm.at[wid, k], w_v)

        _gather(0, rows_a, sga)
        _gather(1, rows_b, sgb)
        _gather(2, rows_c, sgc)

        def _triple(t, _):
            for i in range(3):
                rows, sg, ss = slots[i]
                j = 3 * t + i
                _gather_wait(j, rows, sg)
                _scale(j, rows)
                _scatter(j, rows, ss)
                _ig_add(j, None)
                rows_p, sg_p, ss_p = slots[(i + 2) % 3]
                jp = j - 1

                @pl.when((jp >= 0) & (jp + 3 <= CH - 1))
                def _():
                    _scatter_wait(jp, rows_p, ss_p)
                    _gather(jp + 3, rows_p, sg_p)
            return 0
        lax.fori_loop(0, CH // 3, _triple, 0)

        jl = CH - 1
        _gather_wait(jl, rows_a, sga)
        _scale(jl, rows_a)
        _scatter(jl, rows_a, ssa)
        _ig_add(jl, None)
        _scatter_wait(CH - 3, rows_b, ssb)
        _scatter_wait(CH - 2, rows_c, ssc)
        _scatter_wait(jl, rows_a, ssa)
        pltpu.make_async_copy(ones_sl, ig_s.at[dst_v.at[0]], sai).wait()
        return 0
    lax.fori_loop(0, NCH, _chunk, 0)
    plsc.subcore_barrier()

    pltpu.sync_copy(acc_s.at[pl.ds(lo, ROWS_T), :],
                    part_hbm.at[c, pl.ds(lo, ROWS_T), :])

    @pl.when(c == 0)
    def _():
        pltpu.sync_copy(ig_s.at[pl.ds(lo, 640)], ig0_hbm.at[pl.ds(lo, 640)])

    @pl.when(c == 1)
    def _():
        pltpu.sync_copy(ig_s.at[pl.ds(lo, 640)], ig1_hbm.at[pl.ds(lo, 640)])


def _prescale_body(feat_ref, og0_ref, og1_ref, out_ref):
    d = og0_ref[:, 0] + og1_ref[:, 0]
    ns = lax.rsqrt(jnp.maximum(d, 1.0))
    out_ref[:, :] = feat_ref[:, :] * ns[:, None]


def _tc_prescale(feat, og0, og1):
    blk = 5000
    return pl.pallas_call(
        _prescale_body,
        grid=(N_NODES // blk,),
        in_specs=[
            pl.BlockSpec((blk, F), lambda i: (i, 0)),
            pl.BlockSpec((blk, 1), lambda i: (i, 0)),
            pl.BlockSpec((blk, 1), lambda i: (i, 0)),
        ],
        out_specs=pl.BlockSpec((blk, F), lambda i: (i, 0)),
        out_shape=jax.ShapeDtypeStruct((N_NODES, F), jnp.float32),
    )(feat, og0.reshape(N_PAD, 1), og1.reshape(N_PAD, 1))


def _final_body(p_ref, w_ref, b_ref, ig0_ref, ig1_ref, out_ref):
    acc = p_ref[0, :, :] + p_ref[1, :, :]
    r = jnp.dot(acc, w_ref[:, :], preferred_element_type=jnp.float32)
    d = ig0_ref[:, 0] + ig1_ref[:, 0]
    nd = lax.rsqrt(jnp.maximum(d, 1.0))
    out_ref[:, :] = r * nd[:, None] + b_ref[:, :]


def _tc_final(part, W, b, ig0, ig1):
    blk = 5000
    return pl.pallas_call(
        _final_body,
        grid=(N_NODES // blk,),
        in_specs=[
            pl.BlockSpec((NC, blk, F), lambda i: (0, i, 0)),
            pl.BlockSpec((F, F), lambda i: (0, 0)),
            pl.BlockSpec((1, F), lambda i: (0, 0)),
            pl.BlockSpec((blk, 1), lambda i: (i, 0)),
            pl.BlockSpec((blk, 1), lambda i: (i, 0)),
        ],
        out_specs=pl.BlockSpec((blk, F), lambda i: (i, 0)),
        out_shape=jax.ShapeDtypeStruct((N_NODES, F), jnp.float32),
    )(part, W, b, ig0.reshape(N_PAD, 1), ig1.reshape(N_PAD, 1))


@jax.jit
def kernel(feat, edge_index, edge_weight, W, b):
    src = edge_index[0].astype(jnp.int32)
    dst = edge_index[1].astype(jnp.int32)
    w2 = edge_weight

    og0, og1 = _sc_degrees(src.reshape(NW, NB_W, EB))

    fs = _tc_prescale(feat, og0, og1)
    part, ig0, ig1 = _sc_scatter(fs,
                                 src.reshape(NW, NCH, CH, EB),
                                 dst.reshape(NW, NCH, CH, EB),
                                 w2.reshape(NW, NCH, CH, EB))
    return _tc_final(part, W, b.reshape(1, F), ig0, ig1)
